# Initial kernel scaffold; baseline (speedup 1.0000x reference)
import jax
import jax.numpy as jnp
from jax.experimental import pallas as pl


def kernel(length_q, length_k, embeddings_table):
    raise NotImplementedError("write your pallas kernel here")


# speedup vs baseline: 8.2647x; 8.2647x over previous
import jax
import jax.numpy as jnp
from jax.experimental import pallas as pl
from jax.experimental.pallas import tpu as pltpu

_L = 2048
_V = 257
_D = 64
_GROWS = 4352
_SMIN = -(_L + 127)
_SMAX = 128


def _build_g(table):
    def body(tab_ref, g_ref):
        t0 = tab_ref[0:1, :]
        t256 = tab_ref[256:257, :]
        g_ref[0:2048, :] = jnp.broadcast_to(t0, (2048, _D))
        g_ref[2048:2304, :] = tab_ref[1:257, :]
        g_ref[2304:_GROWS, :] = jnp.broadcast_to(t256, (_GROWS - 2304, _D))

    return pl.pallas_call(
        body,
        out_shape=jax.ShapeDtypeStruct((_GROWS, _D), jnp.float32),
    )(table)


def _expand(delta_arr, g):
    R = 8

    def body(delta_ref, g_ref, out_ref):
        i0 = pl.program_id(0) * R
        delta = delta_ref[0]
        for r in range(R):
            i = i0 + r
            start = jnp.clip(delta - i, _SMIN, _SMAX) + (-_SMIN)
            out_ref[r, :, :] = g_ref[pl.ds(start, _L), :]

    return pl.pallas_call(
        body,
        grid=(_L // R,),
        in_specs=[
            pl.BlockSpec(memory_space=pltpu.SMEM),
            pl.BlockSpec((_GROWS, _D), lambda i: (0, 0)),
        ],
        out_specs=pl.BlockSpec((R, _L, _D), lambda i: (i, 0, 0)),
        out_shape=jax.ShapeDtypeStruct((_L, _L, _D), jnp.float32),
    )(delta_arr, g)


def kernel(length_q, length_k, embeddings_table):
    delta = (jnp.asarray(length_k, jnp.int32)
             - jnp.asarray(length_q, jnp.int32)).reshape(1)
    g = _build_g(embeddings_table)
    return _expand(delta, g)
